# baseline (speedup 1.0000x reference)
import functools

import numpy as np
import jax
import jax.numpy as jnp
from jax import lax
from jax.experimental import pallas as pl
from jax.experimental.pallas import tpu as pltpu
from jax.experimental.pallas import tpu_sc as plsc

B, K, H = 64, 1024, 256
ESS_THRESHOLD = 0.5 * K
CH = 1024
NCHUNK = K // CH
TJ = 128
NB = 16
TINY = np.float32(np.finfo(np.float32).tiny)

_K1 = 2247515013
_K2 = 2545468385
_KS2 = (_K1 ^ _K2 ^ 0x1BD11BDA) & 0xFFFFFFFF


def _u(v):
    return jnp.uint32(v & 0xFFFFFFFF)


def _rotl(x, r):
    return (x << _u(r)) | (x >> _u(32 - r))


def _threefry_bits(x1):
    x0 = jnp.full(x1.shape, _u(_K1), jnp.uint32)

    def rounds(x0, x1, rs):
        for r in rs:
            x0 = x0 + x1
            x1 = _rotl(x1, r)
            x1 = x0 ^ x1
        return x0, x1

    R0 = (13, 15, 26, 6)
    R1 = (17, 29, 16, 24)
    x0, x1 = rounds(x0, x1, R0)
    x0 = x0 + _u(_K2); x1 = x1 + _u(_KS2 + 1)
    x0, x1 = rounds(x0, x1, R1)
    x0 = x0 + _u(_KS2); x1 = x1 + _u(_K1 + 2)
    x0, x1 = rounds(x0, x1, R0)
    x0 = x0 + _u(_K1); x1 = x1 + _u(_K2 + 3)
    x0, x1 = rounds(x0, x1, R1)
    x0 = x0 + _u(_K2); x1 = x1 + _u(_KS2 + 4)
    x0, x1 = rounds(x0, x1, R0)
    x0 = x0 + _u(_KS2); x1 = x1 + _u(_K1 + 5)
    return x0 ^ x1


def _tc_body(off, lw_full_ref, lw_col_ref, anc_ref, nlw_ref, flag_ref):
    pid = pl.program_id(0)

    @pl.when(pid == 0)
    def _():
        lw = lw_full_ref[...]
        m1 = jnp.max(lw, axis=1, keepdims=True)
        s1 = jnp.sum(jnp.exp(lw - m1), axis=1, keepdims=True)
        ln = lw - (jnp.log(s1) + m1)
        x2 = 2.0 * ln
        m2 = jnp.max(x2, axis=1, keepdims=True)
        s2 = jnp.sum(jnp.exp(x2 - m2), axis=1, keepdims=True)
        ess = jnp.exp(-(jnp.log(s2) + m2))
        n = jnp.sum((ess < ESS_THRESHOLD).astype(jnp.int32))
        flag_ref[0] = (n > 0).astype(jnp.int32)

    anyv = flag_ref[0] != 0
    kk = lax.broadcasted_iota(jnp.uint32, (CH, TJ), 0) * _u(65536)
    lane = lax.broadcasted_iota(jnp.uint32, (CH, TJ), 1)
    cnt00 = kk + lane
    lane_i = lax.broadcasted_iota(jnp.int32, (CH, TJ), 1)
    kio = lax.broadcasted_iota(jnp.int32, (K, 1), 0)

    def batch(bi, carry):
        bb = off + pid * NB + bi
        lw_row = lw_full_ref[pl.ds(bb, 1), :]
        prop = 0.5 * jnp.exp(lw_row) + np.float32(0.5 * (1.0 / K))
        logits = jnp.log(prop + np.float32(1e-10))
        d = lw_row - logits

        boff = lax.convert_element_type(bb * 1024, jnp.uint32)
        m = None
        for t in range(K // TJ):
            bits = _threefry_bits(cnt00 + (boff + _u(t * TJ + _K2)))
            fb = (bits >> _u(9)) | _u(0x3F800000)
            f = lax.bitcast_convert_type(fb, jnp.float32) - 1.0
            u = jnp.maximum(f, TINY)
            y = jnp.log(-jnp.log(u)) - logits[:, t * TJ:(t + 1) * TJ]
            d_t = d[:, t * TJ:(t + 1) * TJ]
            if m is None:
                m, jidx, dgr = y, lane_i, jnp.broadcast_to(d_t, (CH, TJ))
            else:
                lt = y < m
                m = jnp.where(lt, y, m)
                jidx = jnp.where(lt, lane_i + t * TJ, jidx)
                dgr = jnp.where(lt, d_t, dgr)
        mfin = jnp.min(m, axis=1, keepdims=True)
        a = jnp.min(jnp.where(m == mfin, jidx, K), axis=1, keepdims=True)
        dg = jnp.sum(jnp.where(jidx == a, dgr, 0.0), axis=1, keepdims=True)

        m3 = jnp.max(dg)
        lse = jnp.log(jnp.sum(jnp.exp(dg - m3))) + m3
        lw_col = lw_col_ref[pl.ds(bi, 1)].reshape(K, 1)
        nlw_ref[pl.ds(bi, 1)] = jnp.where(anyv, dg - lse,
                                          lw_col).reshape(1, K, 1)
        anc_ref[pl.ds(bi, 1)] = (jnp.where(anyv, a, kio)
                                 + bb * 1024).reshape(1, K, 1)
        return carry

    lax.fori_loop(0, NB, batch, 0)


HB = B // 2


def _make_tc_half(off):
    return pl.pallas_call(
        functools.partial(_tc_body, off),
        grid=(HB // NB,),
        in_specs=[
            pl.BlockSpec((B, K), lambda i: (0, 0)),
            pl.BlockSpec((NB, K, 1), lambda i: (i, 0, 0)),
        ],
        out_specs=[
            pl.BlockSpec((NB, K, 1), lambda i: (i, 0, 0)),
            pl.BlockSpec((NB, K, 1), lambda i: (i, 0, 0)),
        ],
        out_shape=[
            jax.ShapeDtypeStruct((HB, K, 1), jnp.int32),
            jax.ShapeDtypeStruct((HB, K, 1), jnp.float32),
        ],
        scratch_shapes=[
            pltpu.SMEM((1,), jnp.int32),
        ],
        compiler_params=pltpu.CompilerParams(
            dimension_semantics=("arbitrary",)),
    )


_tc_half0 = _make_tc_half(0)
_tc_half1 = _make_tc_half(HB)


_NC, _NS = 2, 16
_NW = _NC * _NS
_GCH = 128


def _sc_gather_body(rpw, table, idx, out, idx_v, buf0, buf1, sem0, sem1):
    wid = lax.axis_index("s") * _NC + lax.axis_index("c")
    base = wid * rpw
    pltpu.sync_copy(idx.at[pl.ds(base, rpw)], idx_v)
    bufs = (buf0, buf1)
    sems = (sem0, sem1)
    prev = None
    for i in range(rpw // _GCH):
        cp = pltpu.async_copy(
            table.at[idx_v.at[pl.ds(i * _GCH, _GCH)]], bufs[i % 2], sems[i % 2])
        if prev is not None:
            pcp, pbuf, poff = prev
            pcp.wait()
            pltpu.sync_copy(pbuf, out.at[pl.ds(poff, _GCH)])
        prev = (cp, bufs[i % 2], base + i * _GCH)
    pcp, pbuf, poff = prev
    pcp.wait()
    pltpu.sync_copy(pbuf, out.at[pl.ds(poff, _GCH)])


@functools.lru_cache(maxsize=None)
def _make_sc_gather(nrows):
    rpw = nrows // _NW
    return functools.partial(
        pl.kernel,
        out_type=jax.ShapeDtypeStruct((nrows, H), jnp.float32),
        mesh=plsc.VectorSubcoreMesh(core_axis_name="c", subcore_axis_name="s",
                                    num_cores=_NC, num_subcores=_NS),
        scratch_types=[
            pltpu.VMEM((rpw,), jnp.int32),
            pltpu.VMEM((_GCH, H), jnp.float32),
            pltpu.VMEM((_GCH, H), jnp.float32),
            pltpu.SemaphoreType.DMA,
            pltpu.SemaphoreType.DMA,
        ],
    )(functools.partial(_sc_gather_body, rpw))


def kernel(particles, log_weights):
    lw3 = log_weights.reshape(B, K, 1)
    table = particles.reshape(B * K, H)
    gather = _make_sc_gather(HB * K)
    anc0, nlw0 = _tc_half0(log_weights, lw3[:HB])
    g0 = gather(table, anc0.reshape(HB * K))
    anc1, nlw1 = _tc_half1(log_weights, lw3[HB:])
    g1 = gather(table, anc1.reshape(HB * K))
    out = jnp.concatenate([g0, g1], axis=0).reshape(B, K, H)
    nlw = jnp.concatenate([nlw0, nlw1], axis=0).reshape(B, K)
    return out, nlw


# speedup vs baseline: 1.0468x; 1.0468x over previous
import functools

import numpy as np
import jax
import jax.numpy as jnp
from jax import lax
from jax.experimental import pallas as pl
from jax.experimental.pallas import tpu as pltpu
from jax.experimental.pallas import tpu_sc as plsc

B, K, H = 64, 1024, 256
ESS_THRESHOLD = 0.5 * K
CH = 1024
NCHUNK = K // CH
TJ = 128
NB = 16
TINY = np.float32(np.finfo(np.float32).tiny)

_K1 = 2247515013
_K2 = 2545468385
_KS2 = (_K1 ^ _K2 ^ 0x1BD11BDA) & 0xFFFFFFFF


def _u(v):
    return jnp.uint32(v & 0xFFFFFFFF)


def _rotl(x, r):
    return (x << _u(r)) | (x >> _u(32 - r))


def _threefry_bits(x1):
    x0 = jnp.full(x1.shape, _u(_K1), jnp.uint32)

    def rounds(x0, x1, rs):
        for r in rs:
            x0 = x0 + x1
            x1 = _rotl(x1, r)
            x1 = x0 ^ x1
        return x0, x1

    R0 = (13, 15, 26, 6)
    R1 = (17, 29, 16, 24)
    x0, x1 = rounds(x0, x1, R0)
    x0 = x0 + _u(_K2); x1 = x1 + _u(_KS2 + 1)
    x0, x1 = rounds(x0, x1, R1)
    x0 = x0 + _u(_KS2); x1 = x1 + _u(_K1 + 2)
    x0, x1 = rounds(x0, x1, R0)
    x0 = x0 + _u(_K1); x1 = x1 + _u(_K2 + 3)
    x0, x1 = rounds(x0, x1, R1)
    x0 = x0 + _u(_K2); x1 = x1 + _u(_KS2 + 4)
    x0, x1 = rounds(x0, x1, R0)
    x0 = x0 + _u(_KS2); x1 = x1 + _u(_K1 + 5)
    return x0 ^ x1


def _tc_body(lw_full_ref, lw_col_ref, anc_ref, nlw_ref, flag_ref):
    pid = pl.program_id(0)

    @pl.when(pid == 0)
    def _():
        lw = lw_full_ref[...]
        m1 = jnp.max(lw, axis=1, keepdims=True)
        s1 = jnp.sum(jnp.exp(lw - m1), axis=1, keepdims=True)
        ln = lw - (jnp.log(s1) + m1)
        x2 = 2.0 * ln
        m2 = jnp.max(x2, axis=1, keepdims=True)
        s2 = jnp.sum(jnp.exp(x2 - m2), axis=1, keepdims=True)
        ess = jnp.exp(-(jnp.log(s2) + m2))
        n = jnp.sum((ess < ESS_THRESHOLD).astype(jnp.int32))
        flag_ref[0] = (n > 0).astype(jnp.int32)

    anyv = flag_ref[0] != 0
    kk = lax.broadcasted_iota(jnp.uint32, (CH, TJ), 0) * _u(65536)
    lane = lax.broadcasted_iota(jnp.uint32, (CH, TJ), 1)
    cnt00 = kk + lane
    lane_i = lax.broadcasted_iota(jnp.int32, (CH, TJ), 1)
    kio = lax.broadcasted_iota(jnp.int32, (K, 1), 0)

    def batch(bi, carry):
        bb = pid * NB + bi
        lw_row = lw_full_ref[pl.ds(bb, 1), :]
        prop = 0.5 * jnp.exp(lw_row) + np.float32(0.5 * (1.0 / K))
        logits = jnp.log(prop + np.float32(1e-10))
        d = lw_row - logits

        boff = lax.convert_element_type(bb * 1024, jnp.uint32)
        m = None
        for t in range(K // TJ):
            bits = _threefry_bits(cnt00 + (boff + _u(t * TJ + _K2)))
            fb = (bits >> _u(9)) | _u(0x3F800000)
            f = lax.bitcast_convert_type(fb, jnp.float32) - 1.0
            u = jnp.maximum(f, TINY)
            y = jnp.log(-jnp.log(u)) - logits[:, t * TJ:(t + 1) * TJ]
            d_t = d[:, t * TJ:(t + 1) * TJ]
            if m is None:
                m, jidx, dgr = y, lane_i, jnp.broadcast_to(d_t, (CH, TJ))
            else:
                lt = y < m
                m = jnp.where(lt, y, m)
                jidx = jnp.where(lt, lane_i + t * TJ, jidx)
                dgr = jnp.where(lt, d_t, dgr)
        mfin = jnp.min(m, axis=1, keepdims=True)
        a = jnp.min(jnp.where(m == mfin, jidx, K), axis=1, keepdims=True)
        dg = jnp.sum(jnp.where(jidx == a, dgr, 0.0), axis=1, keepdims=True)

        m3 = jnp.max(dg)
        lse = jnp.log(jnp.sum(jnp.exp(dg - m3))) + m3
        lw_col = lw_col_ref[pl.ds(bi, 1)].reshape(K, 1)
        nlw_ref[pl.ds(bi, 1)] = jnp.where(anyv, dg - lse,
                                          lw_col).reshape(1, K, 1)
        anc_ref[pl.ds(bi, 1)] = (jnp.where(anyv, a, kio)
                                 + bb * 1024).reshape(1, K, 1)
        return carry

    lax.fori_loop(0, NB, batch, 0)


_tc_call = pl.pallas_call(
    _tc_body,
    grid=(B // NB,),
    in_specs=[
        pl.BlockSpec((B, K), lambda i: (0, 0)),
        pl.BlockSpec((NB, K, 1), lambda i: (i, 0, 0)),
    ],
    out_specs=[
        pl.BlockSpec((NB, K, 1), lambda i: (i, 0, 0)),
        pl.BlockSpec((NB, K, 1), lambda i: (i, 0, 0)),
    ],
    out_shape=[
        jax.ShapeDtypeStruct((B, K, 1), jnp.int32),
        jax.ShapeDtypeStruct((B, K, 1), jnp.float32),
    ],
    scratch_shapes=[
        pltpu.SMEM((1,), jnp.int32),
    ],
    compiler_params=pltpu.CompilerParams(
        dimension_semantics=("arbitrary",)),
)


_NC, _NS = 2, 16
_NW = _NC * _NS
_RPW = (B * K) // _NW
_GCH = 128
_NG = _RPW // _GCH


def _sc_gather_body(table, idx, out, idx_v, buf0, buf1, sem0, sem1):
    wid = lax.axis_index("s") * _NC + lax.axis_index("c")
    base = wid * _RPW
    pltpu.sync_copy(idx.at[pl.ds(base, _RPW)], idx_v)
    bufs = (buf0, buf1)
    sems = (sem0, sem1)
    prev = None
    for i in range(_NG):
        cp = pltpu.async_copy(
            table.at[idx_v.at[pl.ds(i * _GCH, _GCH)]], bufs[i % 2], sems[i % 2])
        if prev is not None:
            pcp, pbuf, poff = prev
            pcp.wait()
            pltpu.sync_copy(pbuf, out.at[pl.ds(poff, _GCH)])
        prev = (cp, bufs[i % 2], base + i * _GCH)
    pcp, pbuf, poff = prev
    pcp.wait()
    pltpu.sync_copy(pbuf, out.at[pl.ds(poff, _GCH)])


@functools.lru_cache(maxsize=None)
def _make_sc_gather():
    return functools.partial(
        pl.kernel,
        out_type=jax.ShapeDtypeStruct((B * K, H), jnp.float32),
        mesh=plsc.VectorSubcoreMesh(core_axis_name="c", subcore_axis_name="s",
                                    num_cores=_NC, num_subcores=_NS),
        scratch_types=[
            pltpu.VMEM((_RPW,), jnp.int32),
            pltpu.VMEM((_GCH, H), jnp.float32),
            pltpu.VMEM((_GCH, H), jnp.float32),
            pltpu.SemaphoreType.DMA,
            pltpu.SemaphoreType.DMA,
        ],
    )(_sc_gather_body)


def kernel(particles, log_weights):
    anc3, nlw3 = _tc_call(log_weights, log_weights.reshape(B, K, 1))
    table = particles.reshape(B * K, H)
    out_flat = _make_sc_gather()(table, anc3.reshape(B * K))
    return out_flat.reshape(B, K, H), nlw3.reshape(B, K)
